# Initial kernel scaffold; baseline (speedup 1.0000x reference)
import jax
import jax.numpy as jnp
from jax.experimental import pallas as pl


def kernel(x):
    raise NotImplementedError("write your pallas kernel here")


# speedup vs baseline: 4381.7686x; 4381.7686x over previous
import jax
import jax.numpy as jnp
from jax.experimental import pallas as pl
from jax.experimental.pallas import tpu as pltpu

_N = 1024
_P1 = 160
_P2 = 80
_BG_AREA = 321 ** 2 - 161 ** 2
_FRONT_DIV = (161 ** 2) * 1.8
_SCALE = float(_BG_AREA / _FRONT_DIV)


def _cfar_kernel(x_ref, o_ref):
    x = x_ref[0, 0]
    i = jax.lax.broadcasted_iota(jnp.int32, (_N, _N), 0)
    j = jax.lax.broadcasted_iota(jnp.int32, (_N, _N), 1)
    d = jnp.abs(i - j)
    b1 = jnp.where(d <= _P1, jnp.float32(1.0), jnp.float32(0.0))
    b2 = jnp.where(d <= _P2, jnp.float32(1.0), jnp.float32(0.0))
    y1 = jnp.dot(b1, x, preferred_element_type=jnp.float32)
    y2 = jnp.dot(b2, x, preferred_element_type=jnp.float32)
    allsum = jnp.dot(y1, b1, preferred_element_type=jnp.float32)
    front = jnp.dot(y2, b2, preferred_element_type=jnp.float32)
    o_ref[0, 0] = front * (_SCALE / (allsum - front))


def kernel(x):
    return pl.pallas_call(
        _cfar_kernel,
        out_shape=jax.ShapeDtypeStruct((4, 1, _N, _N), jnp.float32),
        grid=(4,),
        in_specs=[pl.BlockSpec((1, 1, _N, _N), lambda b: (b, 0, 0, 0))],
        out_specs=pl.BlockSpec((1, 1, _N, _N), lambda b: (b, 0, 0, 0)),
        compiler_params=pltpu.CompilerParams(
            dimension_semantics=("parallel",),
            vmem_limit_bytes=100 * 1024 * 1024,
        ),
        name="cfar_banded_matmul",
    )(x)
